# baseline (speedup 1.0000x reference)
import jax
import jax.numpy as jnp
from jax.experimental import pallas as pl
from jax.experimental.pallas import tpu as pltpu

_BN = 2048


def _lmhead_block(x_ref, w_ref, s_ref, o_ref):
    w = w_ref[...].astype(jnp.bfloat16)
    acc = jax.lax.dot_general(
        x_ref[...],
        w,
        dimension_numbers=(((1,), (1,)), ((), ())),
        preferred_element_type=jnp.float32,
    )
    o_ref[...] = acc * s_ref[...]


def kernel(x, quant_weight, scales):
    b, s, k = x.shape
    n = quant_weight.shape[0]
    m = b * s
    xm = x.reshape(m, k).astype(jnp.bfloat16)
    s2 = scales.reshape(1, n)
    out = pl.pallas_call(
        _lmhead_block,
        grid=(pl.cdiv(n, _BN),),
        in_specs=[
            pl.BlockSpec((m, k), lambda i: (0, 0)),
            pl.BlockSpec((_BN, k), lambda i: (i, 0)),
            pl.BlockSpec((1, _BN), lambda i: (0, i)),
        ],
        out_specs=pl.BlockSpec((m, _BN), lambda i: (0, i)),
        out_shape=jax.ShapeDtypeStruct((m, n), jnp.float32),
        compiler_params=pltpu.CompilerParams(
            dimension_semantics=("arbitrary",),
            vmem_limit_bytes=100 * 1024 * 1024,
        ),
    )(xm, quant_weight, s2)
    return out.reshape(b, s, n)


# speedup vs baseline: 1.0011x; 1.0011x over previous
import jax
import jax.numpy as jnp
from jax.experimental import pallas as pl
from jax.experimental.pallas import tpu as pltpu

_BN = 2048


def _lmhead_block(x_ref, w_ref, s_ref, o_ref):
    w = w_ref[...].astype(jnp.bfloat16)
    acc = jax.lax.dot_general(
        x_ref[...],
        w,
        dimension_numbers=(((1,), (1,)), ((), ())),
        preferred_element_type=jnp.float32,
    )
    o_ref[...] = acc * s_ref[...]


def kernel(x, quant_weight, scales):
    b, s, k = x.shape
    n = quant_weight.shape[0]
    m = b * s
    xm = x.reshape(m, k).astype(jnp.bfloat16)
    s2 = scales.reshape(1, n)
    out = pl.pallas_call(
        _lmhead_block,
        grid=(pl.cdiv(n, _BN),),
        in_specs=[
            pl.BlockSpec((m, k), lambda i: (0, 0)),
            pl.BlockSpec((_BN, k), lambda i: (i, 0)),
            pl.BlockSpec((1, _BN), lambda i: (0, i)),
        ],
        out_specs=pl.BlockSpec((m, _BN), lambda i: (0, i)),
        out_shape=jax.ShapeDtypeStruct((m, n), jnp.float32),
        compiler_params=pltpu.CompilerParams(
            dimension_semantics=("arbitrary",),
            allow_input_fusion=[True, False, False],
            vmem_limit_bytes=100 * 1024 * 1024,
        ),
    )(xm, quant_weight, s2)
    return out.reshape(b, s, n)
